# baseline (speedup 1.0000x reference)
import functools

import jax
import jax.numpy as jnp
from jax.experimental import pallas as pl
from jax.experimental.pallas import tpu as pltpu

D_OUT = 8
LOG2E = 1.4426950408889634


def _proj_kernel(x_ref, w1_ref, b1_ref, w2_ref, asrc_ref, adst_ref,
                 gt_ref, gtf_ref, a_s_ref, a_sr_ref, ad2_ref, ad3_ref):
    ht = jax.lax.dot_general(w1_ref[...], x_ref[...], (((1,), (1,)), ((), ())),
                             preferred_element_type=jnp.float32)
    ht = ht + b1_ref[...]
    ht = jnp.where(ht >= 0, ht, 0.01 * ht)
    gt = jax.lax.dot_general(w2_ref[...], ht, (((1,), (0,)), ((), ())),
                             preferred_element_type=jnp.float32)
    gt_ref[:D_OUT, :] = gt.astype(jnp.bfloat16)
    gt_ref[D_OUT:, :] = jnp.ones_like(gt_ref[D_OUT:, :])
    gtf_ref[...] = gt
    g = jax.lax.dot_general(ht, w2_ref[...], (((0,), (1,)), ((), ())),
                            preferred_element_type=jnp.float32)
    a_s = LOG2E * jax.lax.dot_general(g, asrc_ref[...], (((1,), (1,)), ((), ())),
                                      preferred_element_type=jnp.float32)
    a_s_ref[...] = a_s
    a_sr = LOG2E * jax.lax.dot_general(asrc_ref[...], gt, (((1,), (0,)), ((), ())),
                                       preferred_element_type=jnp.float32)
    a_sr_ref[...] = a_sr
    a_d = LOG2E * jax.lax.dot_general(adst_ref[...], gt, (((1,), (0,)), ((), ())),
                                      preferred_element_type=jnp.float32)
    t = jnp.max(a_s) + a_d
    m = jnp.maximum(t, 0.2 * t)
    ad2_ref[...] = a_d - m
    ad3_ref[...] = 0.2 * a_d - m


def _attn_kernel(adj0_ref, adj1_ref, gt0_ref, gt1_ref, a_s0_ref, a_s1_ref,
                 a_sr_ref, ad2_ref, ad3_ref, gtf_ref, bias_ref, out_ref,
                 acc_ref, diag_ref, *, ib, ni):
    i = pl.program_id(0)

    @pl.when(i == 0)
    def _init():
        acc_ref[...] = jnp.zeros_like(acc_ref)

    ad2 = ad2_ref[...]
    ad3 = ad3_ref[...]
    d0 = (jax.lax.broadcasted_iota(jnp.int32, (ib, ib), 0)
          - jax.lax.broadcasted_iota(jnp.int32, (ib, ib), 1))

    def _strip(adj_ref, gt_ref, a_s_ref, base):
        a = adj_ref[...]
        asv = a_s_ref[...]
        u = jnp.maximum(asv + ad2, 0.2 * asv + ad3)
        pf = jnp.exp2(u)
        p = jnp.where(a != 0, pf, 0.0).astype(jnp.bfloat16)
        acc_ref[...] = acc_ref[...] + jax.lax.dot_general(
            gt_ref[...], p, (((1,), (0,)), ((), ())),
            preferred_element_type=jnp.float32)
        asub = adj_ref[:, pl.ds(base * ib, ib)]
        diag_ref[:, pl.ds(base * ib, ib)] = jnp.sum(
            jnp.where(d0 == 0, asub, 0), axis=0, keepdims=True)

    _strip(adj0_ref, gt0_ref, a_s0_ref, i)
    _strip(adj1_ref, gt1_ref, a_s1_ref, i + ni)

    @pl.when(i == ni - 1)
    def _fini():
        asr = a_sr_ref[...]
        ud = jnp.maximum(asr + ad2, 0.2 * asr + ad3)
        pfd = jnp.exp2(ud)
        w = jnp.where(diag_ref[...] != 0, 0.0, pfd)
        s = acc_ref[D_OUT:, :] + w
        num = acc_ref[:D_OUT, :] + gtf_ref[...] * w
        o = num / (s + 1e-16) + bias_ref[...]
        out_ref[...] = o.T


@jax.jit
def kernel(x, adj, W1, b1, W2, att_src, att_dst, bias):
    n = x.shape[0]

    gt, gtf, a_s, a_sr, ad2, ad3 = pl.pallas_call(
        _proj_kernel,
        out_shape=(
            jax.ShapeDtypeStruct((D_OUT + 1, n), jnp.bfloat16),
            jax.ShapeDtypeStruct((D_OUT, n), jnp.float32),
            jax.ShapeDtypeStruct((n, 1), jnp.float32),
            jax.ShapeDtypeStruct((1, n), jnp.float32),
            jax.ShapeDtypeStruct((1, n), jnp.float32),
            jax.ShapeDtypeStruct((1, n), jnp.float32),
        ),
    )(x, W1, b1.reshape(-1, 1), W2,
      att_src.reshape(1, -1), att_dst.reshape(1, -1))

    ib = 512
    ni = n // ib // 2

    out = pl.pallas_call(
        functools.partial(_attn_kernel, ib=ib, ni=ni),
        grid=(ni,),
        in_specs=[
            pl.BlockSpec((ib, n), lambda i: (i, 0)),
            pl.BlockSpec((ib, n), lambda i: (i + 4, 0)),
            pl.BlockSpec((D_OUT + 1, ib), lambda i: (0, i)),
            pl.BlockSpec((D_OUT + 1, ib), lambda i: (0, i + 4)),
            pl.BlockSpec((ib, 1), lambda i: (i, 0)),
            pl.BlockSpec((ib, 1), lambda i: (i + 4, 0)),
            pl.BlockSpec((1, n), lambda i: (0, 0)),
            pl.BlockSpec((1, n), lambda i: (0, 0)),
            pl.BlockSpec((1, n), lambda i: (0, 0)),
            pl.BlockSpec((D_OUT, n), lambda i: (0, 0)),
            pl.BlockSpec((D_OUT, 1), lambda i: (0, 0)),
        ],
        out_specs=pl.BlockSpec((n, D_OUT), lambda i: (0, 0)),
        out_shape=jax.ShapeDtypeStruct((n, D_OUT), jnp.float32),
        scratch_shapes=[
            pltpu.VMEM((D_OUT + 1, n), jnp.float32),
            pltpu.VMEM((1, n), jnp.int32),
        ],
    )(adj, adj, gt, gt, a_s, a_s, a_sr, ad2, ad3, gtf, bias.reshape(-1, 1))

    return out


# speedup vs baseline: 1.0163x; 1.0163x over previous
import functools

import jax
import jax.numpy as jnp
from jax.experimental import pallas as pl
from jax.experimental.pallas import tpu as pltpu

D_OUT = 8
LOG2E = 1.4426950408889634


def _proj_kernel(x_ref, w1_ref, b1_ref, w2_ref, asrc_ref, adst_ref,
                 gt_ref, gtf_ref, e1_ref, e2_ref, a_sr_ref, ad2_ref, ad3_ref,
                 f1_ref, f2_ref):
    ht = jax.lax.dot_general(w1_ref[...], x_ref[...], (((1,), (1,)), ((), ())),
                             preferred_element_type=jnp.float32)
    ht = ht + b1_ref[...]
    ht = jnp.where(ht >= 0, ht, 0.01 * ht)
    gt = jax.lax.dot_general(w2_ref[...], ht, (((1,), (0,)), ((), ())),
                             preferred_element_type=jnp.float32)
    gt_ref[:D_OUT, :] = gt.astype(jnp.bfloat16)
    gt_ref[D_OUT:, :] = jnp.ones_like(gt_ref[D_OUT:, :])
    gtf_ref[...] = gt
    g = jax.lax.dot_general(ht, w2_ref[...], (((0,), (1,)), ((), ())),
                            preferred_element_type=jnp.float32)
    a_s = LOG2E * jax.lax.dot_general(g, asrc_ref[...], (((1,), (1,)), ((), ())),
                                      preferred_element_type=jnp.float32)
    e1_ref[...] = jnp.exp2(a_s)
    e2_ref[...] = jnp.exp2(0.2 * a_s)
    a_sr = LOG2E * jax.lax.dot_general(asrc_ref[...], gt, (((1,), (0,)), ((), ())),
                                       preferred_element_type=jnp.float32)
    a_sr_ref[...] = a_sr
    a_d = LOG2E * jax.lax.dot_general(adst_ref[...], gt, (((1,), (0,)), ((), ())),
                                      preferred_element_type=jnp.float32)
    t = jnp.max(a_s) + a_d
    m = jnp.maximum(t, 0.2 * t)
    ad2 = a_d - m
    ad3 = 0.2 * a_d - m
    ad2_ref[...] = ad2
    ad3_ref[...] = ad3
    f1_ref[...] = jnp.exp2(ad2)
    f2_ref[...] = jnp.exp2(ad3)


def _attn_kernel(adj0_ref, adj1_ref, gt0_ref, gt1_ref, e1c0_ref, e1c1_ref,
                 e2c0_ref, e2c1_ref, f1_ref, f2_ref,
                 a_sr_ref, ad2_ref, ad3_ref, gtf_ref, bias_ref, out_ref,
                 acc_ref, diag_ref, *, ib, ni):
    i = pl.program_id(0)

    @pl.when(i == 0)
    def _init():
        acc_ref[...] = jnp.zeros_like(acc_ref)

    f1 = f1_ref[...]
    f2 = f2_ref[...]
    d0 = (jax.lax.broadcasted_iota(jnp.int32, (ib, ib), 0)
          - jax.lax.broadcasted_iota(jnp.int32, (ib, ib), 1))

    def _strip(adj_ref, gt_ref, e1c_ref, e2c_ref, base):
        a = adj_ref[...]
        pf = jnp.maximum(e1c_ref[...] * f1, e2c_ref[...] * f2)
        p = jnp.where(a != 0, pf, 0.0).astype(jnp.bfloat16)
        acc_ref[...] = acc_ref[...] + jax.lax.dot_general(
            gt_ref[...], p, (((1,), (0,)), ((), ())),
            preferred_element_type=jnp.float32)
        asub = adj_ref[:, pl.ds(base * ib, ib)]
        diag_ref[:, pl.ds(base * ib, ib)] = jnp.sum(
            jnp.where(d0 == 0, asub, 0), axis=0, keepdims=True)

    _strip(adj0_ref, gt0_ref, e1c0_ref, e2c0_ref, i)
    _strip(adj1_ref, gt1_ref, e1c1_ref, e2c1_ref, i + ni)

    @pl.when(i == ni - 1)
    def _fini():
        asr = a_sr_ref[...]
        ud = jnp.maximum(asr + ad2_ref[...],
                         0.2 * asr + ad3_ref[...])
        pfd = jnp.exp2(ud)
        w = jnp.where(diag_ref[...] != 0, 0.0, pfd)
        s = acc_ref[D_OUT:, :] + w
        num = acc_ref[:D_OUT, :] + gtf_ref[...] * w
        o = num / (s + 1e-16) + bias_ref[...]
        out_ref[...] = o.T


@jax.jit
def kernel(x, adj, W1, b1, W2, att_src, att_dst, bias):
    n = x.shape[0]

    gt, gtf, e1, e2, a_sr, ad2, ad3, f1, f2 = pl.pallas_call(
        _proj_kernel,
        out_shape=(
            jax.ShapeDtypeStruct((D_OUT + 1, n), jnp.bfloat16),
            jax.ShapeDtypeStruct((D_OUT, n), jnp.float32),
            jax.ShapeDtypeStruct((n, 1), jnp.float32),
            jax.ShapeDtypeStruct((n, 1), jnp.float32),
            jax.ShapeDtypeStruct((1, n), jnp.float32),
            jax.ShapeDtypeStruct((1, n), jnp.float32),
            jax.ShapeDtypeStruct((1, n), jnp.float32),
            jax.ShapeDtypeStruct((1, n), jnp.float32),
            jax.ShapeDtypeStruct((1, n), jnp.float32),
        ),
    )(x, W1, b1.reshape(-1, 1), W2,
      att_src.reshape(1, -1), att_dst.reshape(1, -1))

    ib = 256
    ni = n // ib // 2

    out = pl.pallas_call(
        functools.partial(_attn_kernel, ib=ib, ni=ni),
        grid=(ni,),
        in_specs=[
            pl.BlockSpec((ib, n), lambda i: (i, 0)),
            pl.BlockSpec((ib, n), lambda i: (i + 8, 0)),
            pl.BlockSpec((D_OUT + 1, ib), lambda i: (0, i)),
            pl.BlockSpec((D_OUT + 1, ib), lambda i: (0, i + 8)),
            pl.BlockSpec((ib, 1), lambda i: (i, 0)),
            pl.BlockSpec((ib, 1), lambda i: (i + 8, 0)),
            pl.BlockSpec((ib, 1), lambda i: (i, 0)),
            pl.BlockSpec((ib, 1), lambda i: (i + 8, 0)),
            pl.BlockSpec((1, n), lambda i: (0, 0)),
            pl.BlockSpec((1, n), lambda i: (0, 0)),
            pl.BlockSpec((1, n), lambda i: (0, 0)),
            pl.BlockSpec((1, n), lambda i: (0, 0)),
            pl.BlockSpec((1, n), lambda i: (0, 0)),
            pl.BlockSpec((D_OUT, n), lambda i: (0, 0)),
            pl.BlockSpec((D_OUT, 1), lambda i: (0, 0)),
        ],
        out_specs=pl.BlockSpec((n, D_OUT), lambda i: (0, 0)),
        out_shape=jax.ShapeDtypeStruct((n, D_OUT), jnp.float32),
        scratch_shapes=[
            pltpu.VMEM((D_OUT + 1, n), jnp.float32),
            pltpu.VMEM((1, n), jnp.int32),
        ],
    )(adj, adj, gt, gt, e1, e1, e2, e2, f1, f2, a_sr, ad2, ad3, gtf,
      bias.reshape(-1, 1))

    return out


# speedup vs baseline: 1.0508x; 1.0339x over previous
import functools

import jax
import jax.numpy as jnp
from jax.experimental import pallas as pl
from jax.experimental.pallas import tpu as pltpu

D_OUT = 8
LOG2E = 1.4426950408889634


def _proj_kernel(x_ref, w1_ref, b1_ref, w2_ref, asrc_ref, adst_ref,
                 gt_ref, gtf_ref, e1_ref, e2_ref, a_sr_ref, ad2_ref, ad3_ref,
                 f1_ref, f2_ref):
    ht = jax.lax.dot_general(w1_ref[...], x_ref[...], (((1,), (1,)), ((), ())),
                             preferred_element_type=jnp.float32)
    ht = ht + b1_ref[...]
    ht = jnp.where(ht >= 0, ht, 0.01 * ht)
    gt = jax.lax.dot_general(w2_ref[...], ht, (((1,), (0,)), ((), ())),
                             preferred_element_type=jnp.float32)
    gt_ref[:D_OUT, :] = gt.astype(jnp.bfloat16)
    gt_ref[D_OUT:, :] = jnp.ones_like(gt_ref[D_OUT:, :])
    gtf_ref[...] = gt
    g = jax.lax.dot_general(ht, w2_ref[...], (((0,), (1,)), ((), ())),
                            preferred_element_type=jnp.float32)
    a_s = LOG2E * jax.lax.dot_general(g, asrc_ref[...], (((1,), (1,)), ((), ())),
                                      preferred_element_type=jnp.float32)
    e1_ref[...] = jnp.exp2(a_s)
    e2_ref[...] = jnp.exp2(0.2 * a_s)
    a_sr = LOG2E * jax.lax.dot_general(asrc_ref[...], gt, (((1,), (0,)), ((), ())),
                                       preferred_element_type=jnp.float32)
    a_sr_ref[...] = a_sr
    a_d = LOG2E * jax.lax.dot_general(adst_ref[...], gt, (((1,), (0,)), ((), ())),
                                      preferred_element_type=jnp.float32)
    t = jnp.max(a_s) + a_d
    m = jnp.maximum(t, 0.2 * t)
    ad2 = a_d - m
    ad3 = 0.2 * a_d - m
    ad2_ref[...] = ad2
    ad3_ref[...] = ad3
    f1_ref[...] = jnp.exp2(ad2)
    f2_ref[...] = jnp.exp2(ad3)


def _attn_kernel(adj_hbm, gt0_ref, gt1_ref, e1c0_ref, e1c1_ref,
                 e2c0_ref, e2c1_ref, f1_ref, f2_ref,
                 a_sr_ref, ad2_ref, ad3_ref, gtf_ref, bias_ref, out_ref,
                 acc_ref, diag_ref, abuf, dsem, *, ib, ni):
    i = pl.program_id(0)
    nslots = 4

    def _copy(step, half):
        strip = step + half * ni
        slot = (2 * step + half) % nslots
        return pltpu.make_async_copy(
            adj_hbm.at[pl.ds(strip * ib, ib), :], abuf.at[slot],
            dsem.at[slot])

    @pl.when(i == 0)
    def _init():
        acc_ref[...] = jnp.zeros_like(acc_ref)
        _copy(0, 0).start()
        _copy(0, 1).start()

    @pl.when(i + 1 < ni)
    def _prefetch():
        _copy(i + 1, 0).start()
        _copy(i + 1, 1).start()

    f1 = f1_ref[...]
    f2 = f2_ref[...]
    d0 = (jax.lax.broadcasted_iota(jnp.int32, (ib, ib), 0)
          - jax.lax.broadcasted_iota(jnp.int32, (ib, ib), 1))

    def _strip(half, gt_ref, e1c_ref, e2c_ref):
        _copy(i, half).wait()
        adj_ref = abuf.at[(2 * i + half) % nslots]
        base = i + half * ni
        a = adj_ref[...]
        pf = jnp.maximum(e1c_ref[...] * f1, e2c_ref[...] * f2)
        p = jnp.where(a != 0, pf, 0.0).astype(jnp.bfloat16)
        acc_ref[...] = acc_ref[...] + jax.lax.dot_general(
            gt_ref[...], p, (((1,), (0,)), ((), ())),
            preferred_element_type=jnp.float32)
        asub = adj_ref[:, pl.ds(base * ib, ib)]
        diag_ref[:, pl.ds(base * ib, ib)] = jnp.sum(
            jnp.where(d0 == 0, asub, 0), axis=0, keepdims=True)

    _strip(0, gt0_ref, e1c0_ref, e2c0_ref)
    _strip(1, gt1_ref, e1c1_ref, e2c1_ref)

    @pl.when(i == ni - 1)
    def _fini():
        asr = a_sr_ref[...]
        ud = jnp.maximum(asr + ad2_ref[...],
                         0.2 * asr + ad3_ref[...])
        pfd = jnp.exp2(ud)
        w = jnp.where(diag_ref[...] != 0, 0.0, pfd)
        s = acc_ref[D_OUT:, :] + w
        num = acc_ref[:D_OUT, :] + gtf_ref[...] * w
        o = num / (s + 1e-16) + bias_ref[...]
        out_ref[...] = o.T


@jax.jit
def kernel(x, adj, W1, b1, W2, att_src, att_dst, bias):
    n = x.shape[0]

    gt, gtf, e1, e2, a_sr, ad2, ad3, f1, f2 = pl.pallas_call(
        _proj_kernel,
        out_shape=(
            jax.ShapeDtypeStruct((D_OUT + 1, n), jnp.bfloat16),
            jax.ShapeDtypeStruct((D_OUT, n), jnp.float32),
            jax.ShapeDtypeStruct((n, 1), jnp.float32),
            jax.ShapeDtypeStruct((n, 1), jnp.float32),
            jax.ShapeDtypeStruct((1, n), jnp.float32),
            jax.ShapeDtypeStruct((1, n), jnp.float32),
            jax.ShapeDtypeStruct((1, n), jnp.float32),
            jax.ShapeDtypeStruct((1, n), jnp.float32),
            jax.ShapeDtypeStruct((1, n), jnp.float32),
        ),
    )(x, W1, b1.reshape(-1, 1), W2,
      att_src.reshape(1, -1), att_dst.reshape(1, -1))

    ib = 256
    ni = n // ib // 2

    out = pl.pallas_call(
        functools.partial(_attn_kernel, ib=ib, ni=ni),
        grid=(ni,),
        in_specs=[
            pl.BlockSpec(memory_space=pl.ANY),
            pl.BlockSpec((D_OUT + 1, ib), lambda i: (0, i)),
            pl.BlockSpec((D_OUT + 1, ib), lambda i: (0, i + 8)),
            pl.BlockSpec((ib, 1), lambda i: (i, 0)),
            pl.BlockSpec((ib, 1), lambda i: (i + 8, 0)),
            pl.BlockSpec((ib, 1), lambda i: (i, 0)),
            pl.BlockSpec((ib, 1), lambda i: (i + 8, 0)),
            pl.BlockSpec((1, n), lambda i: (0, 0)),
            pl.BlockSpec((1, n), lambda i: (0, 0)),
            pl.BlockSpec((1, n), lambda i: (0, 0)),
            pl.BlockSpec((1, n), lambda i: (0, 0)),
            pl.BlockSpec((1, n), lambda i: (0, 0)),
            pl.BlockSpec((D_OUT, n), lambda i: (0, 0)),
            pl.BlockSpec((D_OUT, 1), lambda i: (0, 0)),
        ],
        out_specs=pl.BlockSpec((n, D_OUT), lambda i: (0, 0)),
        out_shape=jax.ShapeDtypeStruct((n, D_OUT), jnp.float32),
        scratch_shapes=[
            pltpu.VMEM((D_OUT + 1, n), jnp.float32),
            pltpu.VMEM((1, n), jnp.int32),
            pltpu.VMEM((4, ib, n), jnp.int32),
            pltpu.SemaphoreType.DMA((4,)),
        ],
    )(adj, gt, gt, e1, e1, e2, e2, f1, f2, a_sr, ad2, ad3, gtf,
      bias.reshape(-1, 1))

    return out
